# Initial kernel scaffold; baseline (speedup 1.0000x reference)
import jax
import jax.numpy as jnp
from jax.experimental import pallas as pl


def kernel(x, pos, edge_index, batch, emb_W1, emb_b1, emb_W2, emb_b2, msg_W1, msg_b1, msg_W2, msg_b2, edge_W, edge_b, upd_W1, upd_b1, upd_W2, upd_b2, dec_W1, dec_b1, dec_W2, dec_b2, head_W1, head_b1, head_W2, head_b2):
    raise NotImplementedError("write your pallas kernel here")


# speedup vs baseline: 3.9263x; 3.9263x over previous
import functools

import jax
import jax.numpy as jnp
from jax import lax
from jax.experimental import pallas as pl
from jax.experimental.pallas import tpu as pltpu
from jax.experimental.pallas import tpu_sc as plsc

F32 = jnp.float32


def _prep_body(x_ref, ew1, eb1, ew2, eb2, w1i, w1j, b1, h_ref, a_ref, b_ref):
    xx = x_ref[...]
    t = jnp.maximum(jnp.dot(xx, ew1[...], preferred_element_type=F32) + eb1[...], 0.0)
    h = jnp.dot(t, ew2[...], preferred_element_type=F32) + eb2[...]
    h_ref[...] = h
    a_ref[...] = jnp.dot(h, w1i[...], preferred_element_type=F32) + b1[...]
    b_ref[...] = jnp.dot(h, w1j[...], preferred_element_type=F32)


def _prep(x, ew1, eb1, ew2, eb2, w1i, w1j, b1):
    n, f = x.shape
    h_dim = ew1.shape[1]
    blk = 1000
    grid = n // blk
    wspec = lambda shape: pl.BlockSpec(shape, lambda i: (0, 0))
    return pl.pallas_call(
        _prep_body,
        grid=(grid,),
        in_specs=[
            pl.BlockSpec((blk, f), lambda i: (i, 0)),
            wspec(ew1.shape), wspec((1, h_dim)), wspec(ew2.shape), wspec((1, h_dim)),
            wspec(w1i.shape), wspec(w1j.shape), wspec((1, h_dim)),
        ],
        out_specs=[
            pl.BlockSpec((blk, h_dim), lambda i: (i, 0)),
            pl.BlockSpec((blk, h_dim), lambda i: (i, 0)),
            pl.BlockSpec((blk, h_dim), lambda i: (i, 0)),
        ],
        out_shape=[
            jax.ShapeDtypeStruct((n, h_dim), F32),
            jax.ShapeDtypeStruct((n, h_dim), F32),
            jax.ShapeDtypeStruct((n, h_dim), F32),
        ],
    )(x, ew1, eb1.reshape(1, -1), ew2, eb2.reshape(1, -1), w1i, w1j, b1.reshape(1, -1))


_CK = 80


def _sc_gather_body(a_hbm, b_hbm, pos_hbm, src_hbm, dst_hbm, w1c_hbm,
                    t_hbm, c_hbm,
                    posv, w1cv, idxs, idxd, bufa, bufb, distv, cntv,
                    sema, semb):
    num_cores = 2
    num_sub = 16
    cid = lax.axis_index("c")
    sid = lax.axis_index("s")
    wid = sid * num_cores + cid
    e_total = t_hbm.shape[0]
    epw = e_total // (num_cores * num_sub)
    base0 = wid * epw

    pltpu.sync_copy(pos_hbm, posv)
    pltpu.sync_copy(w1c_hbm, w1cv)
    w1 = [w1cv[pl.ds(r * 16, 16)] for r in range(8)]
    n_nodes = a_hbm.shape[0]
    zero16 = jnp.zeros((16,), F32)
    ones16 = jnp.full((16,), 1.0, F32)

    def czl(i, carry):
        cntv[pl.ds(i * 16, 16)] = zero16
        return carry

    lax.fori_loop(0, n_nodes // 16, czl, 0)

    def chunk(c, carry):
        base = base0 + c * _CK
        pltpu.sync_copy(src_hbm.at[pl.ds(base, _CK)], idxs)
        pltpu.sync_copy(dst_hbm.at[pl.ds(base, _CK)], idxd)
        cpa = pltpu.async_copy(a_hbm.at[idxd], bufa, sema)
        cpb = pltpu.async_copy(b_hbm.at[idxs], bufb, semb)

        def dloop(j, carry2):
            d16raw = idxd[pl.ds(j * 16, 16)]
            plsc.addupdate_scatter(cntv, [d16raw], ones16)
            s16 = idxs[pl.ds(j * 16, 16)] * 4
            d16 = d16raw * 4
            dx = plsc.load_gather(posv, [s16]) - plsc.load_gather(posv, [d16])
            dy = plsc.load_gather(posv, [s16 + 1]) - plsc.load_gather(posv, [d16 + 1])
            dz = plsc.load_gather(posv, [s16 + 2]) - plsc.load_gather(posv, [d16 + 2])
            dd = dx * dx + dy * dy + dz * dz
            ii = jnp.int32(0x5F3759DF) - lax.shift_right_logical(
                plsc.bitcast(dd, jnp.int32), 1)
            y = plsc.bitcast(ii, F32)
            y = y * (1.5 - 0.5 * dd * y * y)
            y = y * (1.5 - 0.5 * dd * y * y)
            y = y * (1.5 - 0.5 * dd * y * y)
            distv[pl.ds(j * 16, 16)] = dd * y
            return carry2

        lax.fori_loop(0, _CK // 16, dloop, 0)
        cpa.wait()
        cpb.wait()

        def gloop(g, carry2):
            dvec = distv[pl.ds(g * 16, 16)]
            for lane in range(16):
                dsc = dvec[lane]
                e = g * 16 + lane
                for r in range(8):
                    col = pl.ds(r * 16, 16)
                    tv = bufa[e, col] + bufb[e, col] + dsc * w1[r]
                    bufa[e, col] = jnp.maximum(tv, 0.0)
            return carry2

        lax.fori_loop(0, _CK // 16, gloop, 0)
        pltpu.sync_copy(bufa, t_hbm.at[pl.ds(base, _CK)])
        return carry

    lax.fori_loop(0, epw // _CK, chunk, 0)
    pltpu.sync_copy(cntv, c_hbm.at[pl.ds(wid * n_nodes, n_nodes)])


def _sc_gather(a, b, pos4, src, dst, w1c):
    n, h_dim = a.shape
    e_total = src.shape[0]
    mesh = plsc.VectorSubcoreMesh(core_axis_name="c", subcore_axis_name="s")
    return pl.kernel(
        _sc_gather_body,
        out_type=[
            jax.ShapeDtypeStruct((e_total, h_dim), F32),
            jax.ShapeDtypeStruct((32 * n,), F32),
        ],
        mesh=mesh,
        compiler_params=pltpu.CompilerParams(needs_layout_passes=False),
        scratch_types=[
            pltpu.VMEM((n * 4,), F32),
            pltpu.VMEM((h_dim,), F32),
            pltpu.VMEM((_CK,), jnp.int32),
            pltpu.VMEM((_CK,), jnp.int32),
            pltpu.VMEM((_CK, h_dim), F32),
            pltpu.VMEM((_CK, h_dim), F32),
            pltpu.VMEM((_CK,), F32),
            pltpu.VMEM((n,), F32),
            pltpu.SemaphoreType.DMA,
            pltpu.SemaphoreType.DMA,
        ],
    )(a, b, pos4, src, dst, w1c)


def _edge_body(t_ref, w2, b2, ew, eb, mg_ref):
    t = t_ref[...]
    m = jnp.maximum(jnp.dot(t, w2[...], preferred_element_type=F32) + b2[...], 0.0)
    g = jnp.dot(m, ew[...], preferred_element_type=F32) + eb[...]
    g = 1.0 / (1.0 + jnp.exp(-g))
    mg_ref[...] = m * g


def _edge_mlp(t, w2, b2, ew, eb):
    e_total, h_dim = t.shape
    blk = 3200
    grid = e_total // blk
    wspec = lambda shape: pl.BlockSpec(shape, lambda i: (0, 0))
    return pl.pallas_call(
        _edge_body,
        grid=(grid,),
        in_specs=[
            pl.BlockSpec((blk, h_dim), lambda i: (i, 0)),
            wspec(w2.shape), wspec((1, h_dim)), wspec(ew.shape), wspec((1, 1)),
        ],
        out_specs=pl.BlockSpec((blk, h_dim), lambda i: (i, 0)),
        out_shape=jax.ShapeDtypeStruct((e_total, h_dim), F32),
    )(t, w2, b2.reshape(1, -1), ew, eb.reshape(1, 1))


def _sc_scatter_body(mg_hbm, dst_hbm, s_hbm,
                     acc, bufm, idxd, idxw):
    num_cores = 2
    num_sub = 16
    cid = lax.axis_index("c")
    sid = lax.axis_index("s")
    e_total = mg_hbm.shape[0]
    n = acc.shape[0]
    zrows = _CK
    nchunks = n // zrows
    epc = e_total // num_cores
    ept = epc // num_sub

    zero = jnp.zeros((16,), F32)

    def zloop(r, carry):
        for q in range(8):
            bufm[r, pl.ds(q * 16, 16)] = zero
        return carry

    lax.fori_loop(0, zrows, zloop, 0)

    nround = (nchunks + num_sub - 1) // num_sub
    for q in range(nround):
        ch = sid + q * num_sub

        @pl.when(ch < nchunks)
        def _():
            pltpu.sync_copy(bufm, acc.at[pl.ds(ch * zrows, zrows)])
    plsc.subcore_barrier()

    base0 = cid * epc + sid * ept

    nsub = _CK // 16

    def chunk(c, carry):
        base = base0 + c * _CK
        pltpu.sync_copy(dst_hbm.at[pl.ds(base, _CK)], idxw.at[0])
        pltpu.sync_copy(mg_hbm.at[pl.ds(base, _CK)], bufm)
        for k in range(nsub):
            idxd[k, pl.ds(0, 16)] = idxw[0, pl.ds(k * 16, 16)]
        for k in range(nsub):
            pltpu.sync_copy(bufm.at[pl.ds(k * 16, 16)],
                            acc.at[idxd.at[k]], add=True)
        return carry

    lax.fori_loop(0, ept // _CK, chunk, 0)
    plsc.subcore_barrier()

    for q in range(nround):
        ch = sid + q * num_sub

        @pl.when(ch < nchunks)
        def _():
            row = ch * zrows
            pltpu.sync_copy(acc.at[pl.ds(row, zrows)], s_hbm.at[pl.ds(cid * n + row, zrows)])


def _sc_scatter(mg, dst, n):
    e_total, h_dim = mg.shape
    mesh = plsc.VectorSubcoreMesh(core_axis_name="c", subcore_axis_name="s")
    return pl.kernel(
        _sc_scatter_body,
        out_type=jax.ShapeDtypeStruct((2 * n, h_dim), F32),
        mesh=mesh,
        compiler_params=pltpu.CompilerParams(needs_layout_passes=False),
        scratch_types=[
            pltpu.VMEM_SHARED((n, h_dim), F32),
            pltpu.VMEM((_CK, h_dim), F32),
            pltpu.VMEM((_CK // 16, 16), jnp.int32),
            pltpu.VMEM((1, _CK), jnp.int32),
        ],
    )(mg, dst)


def _tail_body(h_ref, s_ref, c_ref, batch_ref,
               uw1, ub1, uw2, ub2, dw1, db1, dw2, db2,
               hw1, hb1, hw2, hb2, out_ref):
    h = h_ref[...]
    n = h.shape[0]
    sfull = s_ref[...]
    s = sfull[:n] + sfull[n:]
    cfull = c_ref[...]
    cnt = lax.dot_general(cfull, jnp.ones((cfull.shape[0], 1), F32),
                          (((0,), (0,)), ((), ())),
                          preferred_element_type=F32)
    agg = s / jnp.maximum(cnt, 1.0)
    upd_in = jnp.concatenate([h, agg], axis=-1)
    t = jnp.maximum(jnp.dot(upd_in, uw1[...], preferred_element_type=F32) + ub1[...], 0.0)
    h = h + jnp.dot(t, uw2[...], preferred_element_type=F32) + ub2[...]
    t = jnp.maximum(jnp.dot(h, dw1[...], preferred_element_type=F32) + db1[...], 0.0)
    hd = jnp.dot(t, dw2[...], preferred_element_type=F32) + db2[...]
    nb = out_ref.shape[0]
    biota = lax.broadcasted_iota(jnp.int32, (nb, 1), 0)
    oht = (biota == batch_ref[...]).astype(F32)
    pooled = jnp.dot(oht, hd, preferred_element_type=F32)
    t = jnp.maximum(jnp.dot(pooled, hw1[...], preferred_element_type=F32) + hb1[...], 0.0)
    out_ref[...] = jnp.dot(t, hw2[...], preferred_element_type=F32) + hb2[...]


def _tail(h, s_part, c_part, batch_row, nb,
          uw1, ub1, uw2, ub2, dw1, db1, dw2, db2, hw1, hb1, hw2, hb2):
    out_dim = hw2.shape[1]
    return pl.pallas_call(
        _tail_body,
        out_shape=jax.ShapeDtypeStruct((nb, out_dim), F32),
    )(h, s_part, c_part, batch_row,
      uw1, ub1.reshape(1, -1), uw2, ub2.reshape(1, -1),
      dw1, db1.reshape(1, -1), dw2, db2.reshape(1, -1),
      hw1, hb1.reshape(1, -1), hw2, hb2.reshape(1, -1))


def kernel(x, pos, edge_index, batch, emb_W1, emb_b1, emb_W2, emb_b2,
           msg_W1, msg_b1, msg_W2, msg_b2, edge_W, edge_b,
           upd_W1, upd_b1, upd_W2, upd_b2, dec_W1, dec_b1, dec_W2, dec_b2,
           head_W1, head_b1, head_W2, head_b2):
    n, _ = x.shape
    h_dim = emb_W1.shape[1]
    nb = 64
    src = edge_index[0].astype(jnp.int32)
    dst = edge_index[1].astype(jnp.int32)
    pos4 = jnp.pad(pos.astype(F32), ((0, 0), (0, 1))).reshape(-1)
    w1i = msg_W1[:h_dim]
    w1j = msg_W1[h_dim:2 * h_dim]
    w1c = msg_W1[2 * h_dim]

    h, a, b = _prep(x, emb_W1, emb_b1, emb_W2, emb_b2, w1i, w1j, msg_b1)
    t, c_flat = _sc_gather(a, b, pos4, src, dst, w1c)
    c_part = c_flat.reshape(32, n)
    mg = _edge_mlp(t, msg_W2, msg_b2, edge_W, edge_b)
    s_part = _sc_scatter(mg, dst, n)
    batch_row = batch.astype(jnp.int32).reshape(1, n)
    return _tail(h, s_part, c_part, batch_row, nb,
                 upd_W1, upd_b1, upd_W2, upd_b2,
                 dec_W1, dec_b1, dec_W2, dec_b2,
                 head_W1, head_b1, head_W2, head_b2)


# speedup vs baseline: 4.3168x; 1.0995x over previous
import functools

import jax
import jax.numpy as jnp
from jax import lax
from jax.experimental import pallas as pl
from jax.experimental.pallas import tpu as pltpu
from jax.experimental.pallas import tpu_sc as plsc

F32 = jnp.float32


def _prep_body(x_ref, ew1, eb1, ew2, eb2, w1i, w1j, b1, h_ref, a_ref, b_ref):
    xx = x_ref[...]
    t = jnp.maximum(jnp.dot(xx, ew1[...], preferred_element_type=F32) + eb1[...], 0.0)
    h = jnp.dot(t, ew2[...], preferred_element_type=F32) + eb2[...]
    h_ref[...] = h
    a_ref[...] = jnp.dot(h, w1i[...], preferred_element_type=F32) + b1[...]
    b_ref[...] = jnp.dot(h, w1j[...], preferred_element_type=F32)


def _prep(x, ew1, eb1, ew2, eb2, w1i, w1j, b1):
    n, f = x.shape
    h_dim = ew1.shape[1]
    blk = 1000
    grid = n // blk
    wspec = lambda shape: pl.BlockSpec(shape, lambda i: (0, 0))
    return pl.pallas_call(
        _prep_body,
        grid=(grid,),
        in_specs=[
            pl.BlockSpec((blk, f), lambda i: (i, 0)),
            wspec(ew1.shape), wspec((1, h_dim)), wspec(ew2.shape), wspec((1, h_dim)),
            wspec(w1i.shape), wspec(w1j.shape), wspec((1, h_dim)),
        ],
        out_specs=[
            pl.BlockSpec((blk, h_dim), lambda i: (i, 0)),
            pl.BlockSpec((blk, h_dim), lambda i: (i, 0)),
            pl.BlockSpec((blk, h_dim), lambda i: (i, 0)),
        ],
        out_shape=[
            jax.ShapeDtypeStruct((n, h_dim), F32),
            jax.ShapeDtypeStruct((n, h_dim), F32),
            jax.ShapeDtypeStruct((n, h_dim), F32),
        ],
    )(x, ew1, eb1.reshape(1, -1), ew2, eb2.reshape(1, -1), w1i, w1j, b1.reshape(1, -1))


_CK = 80


def _sc_gather_body(a_hbm, b_hbm, pos_hbm, src_hbm, dst_hbm, w1c_hbm,
                    t_hbm, c_hbm,
                    posv, w1cv, idxs0, idxs1, idxd0, idxd1,
                    bufa0, bufa1, bufb0, bufb1, distv, cntv,
                    semis0, semis1, semid0, semid1,
                    sema0, sema1, semb0, semb1, semt0, semt1):
    num_cores = 2
    num_sub = 16
    cid = lax.axis_index("c")
    sid = lax.axis_index("s")
    wid = sid * num_cores + cid
    e_total = t_hbm.shape[0]
    epw = e_total // (num_cores * num_sub)
    nchunks = epw // _CK
    base0 = wid * epw

    idxs = [idxs0, idxs1]
    idxd = [idxd0, idxd1]
    bufa = [bufa0, bufa1]
    bufb = [bufb0, bufb1]
    semis = [semis0, semis1]
    semid = [semid0, semid1]
    sema = [sema0, sema1]
    semb = [semb0, semb1]
    semt = [semt0, semt1]

    pltpu.sync_copy(pos_hbm, posv)
    pltpu.sync_copy(w1c_hbm, w1cv)
    w1 = [w1cv[pl.ds(r * 16, 16)] for r in range(8)]
    n_nodes = a_hbm.shape[0]
    zero16 = jnp.zeros((16,), F32)
    ones16 = jnp.full((16,), 1.0, F32)

    def czl(i, carry):
        cntv[pl.ds(i * 16, 16)] = zero16
        return carry

    lax.fori_loop(0, n_nodes // 16, czl, 0)

    def start_idx(base, p):
        pltpu.async_copy(src_hbm.at[pl.ds(base, _CK)], idxs[p], semis[p])
        pltpu.async_copy(dst_hbm.at[pl.ds(base, _CK)], idxd[p], semid[p])

    def do_chunk(c, p, wait_t, start_next):
        base = base0 + c * _CK
        pltpu.make_async_copy(src_hbm.at[pl.ds(base, _CK)], idxs[p], semis[p]).wait()
        pltpu.make_async_copy(dst_hbm.at[pl.ds(base, _CK)], idxd[p], semid[p]).wait()
        if wait_t:
            pltpu.make_async_copy(bufa[p], t_hbm.at[pl.ds(base, _CK)], semt[p]).wait()
        cpa = pltpu.async_copy(a_hbm.at[idxd[p]], bufa[p], sema[p])
        cpb = pltpu.async_copy(b_hbm.at[idxs[p]], bufb[p], semb[p])
        if start_next:
            start_idx(base + _CK, 1 - p)

        def dloop(j, carry2):
            d16raw = idxd[p][pl.ds(j * 16, 16)]
            plsc.addupdate_scatter(cntv, [d16raw], ones16)
            s16 = idxs[p][pl.ds(j * 16, 16)] * 4
            d16 = d16raw * 4
            dx = plsc.load_gather(posv, [s16]) - plsc.load_gather(posv, [d16])
            dy = plsc.load_gather(posv, [s16 + 1]) - plsc.load_gather(posv, [d16 + 1])
            dz = plsc.load_gather(posv, [s16 + 2]) - plsc.load_gather(posv, [d16 + 2])
            dd = dx * dx + dy * dy + dz * dz
            ii = jnp.int32(0x5F3759DF) - lax.shift_right_logical(
                plsc.bitcast(dd, jnp.int32), 1)
            y = plsc.bitcast(ii, F32)
            y = y * (1.5 - 0.5 * dd * y * y)
            y = y * (1.5 - 0.5 * dd * y * y)
            y = y * (1.5 - 0.5 * dd * y * y)
            distv[pl.ds(j * 16, 16)] = dd * y
            return carry2

        lax.fori_loop(0, _CK // 16, dloop, 0)
        cpa.wait()
        cpb.wait()

        def gloop(g, carry2):
            dvec = distv[pl.ds(g * 16, 16)]
            for lane in range(16):
                dsc = dvec[lane]
                e = g * 16 + lane
                for r in range(8):
                    col = pl.ds(r * 16, 16)
                    tv = bufa[p][e, col] + bufb[p][e, col] + dsc * w1[r]
                    bufa[p][e, col] = jnp.maximum(tv, 0.0)
            return carry2

        lax.fori_loop(0, _CK // 16, gloop, 0)
        pltpu.async_copy(bufa[p], t_hbm.at[pl.ds(base, _CK)], semt[p])

    start_idx(base0, 0)
    do_chunk(0, 0, wait_t=False, start_next=True)
    do_chunk(1, 1, wait_t=False, start_next=True)

    def pair(i, carry):
        c = 2 * i
        do_chunk(c, 0, wait_t=True, start_next=True)
        do_chunk(c + 1, 1, wait_t=True, start_next=True)
        return carry

    lax.fori_loop(1, nchunks // 2, pair, 0)
    do_chunk(nchunks - 1, 0, wait_t=True, start_next=False)
    pltpu.make_async_copy(bufa[1], t_hbm.at[pl.ds(base0, _CK)], semt[1]).wait()
    pltpu.make_async_copy(bufa[0], t_hbm.at[pl.ds(base0, _CK)], semt[0]).wait()
    pltpu.sync_copy(cntv, c_hbm.at[pl.ds(wid * n_nodes, n_nodes)])


def _sc_gather(a, b, pos4, src, dst, w1c):
    n, h_dim = a.shape
    e_total = src.shape[0]
    mesh = plsc.VectorSubcoreMesh(core_axis_name="c", subcore_axis_name="s")
    return pl.kernel(
        _sc_gather_body,
        out_type=[
            jax.ShapeDtypeStruct((e_total, h_dim), F32),
            jax.ShapeDtypeStruct((32 * n,), F32),
        ],
        mesh=mesh,
        compiler_params=pltpu.CompilerParams(needs_layout_passes=False),
        scratch_types=[
            pltpu.VMEM((n * 4,), F32),
            pltpu.VMEM((h_dim,), F32),
            pltpu.VMEM((_CK,), jnp.int32),
            pltpu.VMEM((_CK,), jnp.int32),
            pltpu.VMEM((_CK,), jnp.int32),
            pltpu.VMEM((_CK,), jnp.int32),
            pltpu.VMEM((_CK, h_dim), F32),
            pltpu.VMEM((_CK, h_dim), F32),
            pltpu.VMEM((_CK, h_dim), F32),
            pltpu.VMEM((_CK, h_dim), F32),
            pltpu.VMEM((_CK,), F32),
            pltpu.VMEM((n,), F32),
        ] + [pltpu.SemaphoreType.DMA] * 10,
    )(a, b, pos4, src, dst, w1c)


def _edge_body(t_ref, w2, b2, ew, eb, mg_ref):
    t = t_ref[...]
    m = jnp.maximum(jnp.dot(t, w2[...], preferred_element_type=F32) + b2[...], 0.0)
    g = jnp.dot(m, ew[...], preferred_element_type=F32) + eb[...]
    g = 1.0 / (1.0 + jnp.exp(-g))
    mg_ref[...] = m * g


def _edge_mlp(t, w2, b2, ew, eb):
    e_total, h_dim = t.shape
    blk = 3200
    grid = e_total // blk
    wspec = lambda shape: pl.BlockSpec(shape, lambda i: (0, 0))
    return pl.pallas_call(
        _edge_body,
        grid=(grid,),
        in_specs=[
            pl.BlockSpec((blk, h_dim), lambda i: (i, 0)),
            wspec(w2.shape), wspec((1, h_dim)), wspec(ew.shape), wspec((1, 1)),
        ],
        out_specs=pl.BlockSpec((blk, h_dim), lambda i: (i, 0)),
        out_shape=jax.ShapeDtypeStruct((e_total, h_dim), F32),
    )(t, w2, b2.reshape(1, -1), ew, eb.reshape(1, 1))


def _sc_scatter_body(mg_hbm, dst_hbm, s_hbm,
                     acc, bufm, idxd, idxw):
    num_cores = 2
    num_sub = 16
    cid = lax.axis_index("c")
    sid = lax.axis_index("s")
    e_total = mg_hbm.shape[0]
    n = acc.shape[0]
    zrows = _CK
    nchunks = n // zrows
    epc = e_total // num_cores
    ept = epc // num_sub

    zero = jnp.zeros((16,), F32)

    def zloop(r, carry):
        for q in range(8):
            bufm[r, pl.ds(q * 16, 16)] = zero
        return carry

    lax.fori_loop(0, zrows, zloop, 0)

    nround = (nchunks + num_sub - 1) // num_sub
    for q in range(nround):
        ch = sid + q * num_sub

        @pl.when(ch < nchunks)
        def _():
            pltpu.sync_copy(bufm, acc.at[pl.ds(ch * zrows, zrows)])
    plsc.subcore_barrier()

    base0 = cid * epc + sid * ept

    nsub = _CK // 16

    def chunk(c, carry):
        base = base0 + c * _CK
        pltpu.sync_copy(dst_hbm.at[pl.ds(base, _CK)], idxw.at[0])
        pltpu.sync_copy(mg_hbm.at[pl.ds(base, _CK)], bufm)
        for k in range(nsub):
            idxd[k, pl.ds(0, 16)] = idxw[0, pl.ds(k * 16, 16)]
        for k in range(nsub):
            pltpu.sync_copy(bufm.at[pl.ds(k * 16, 16)],
                            acc.at[idxd.at[k]], add=True)
        return carry

    lax.fori_loop(0, ept // _CK, chunk, 0)
    plsc.subcore_barrier()

    for q in range(nround):
        ch = sid + q * num_sub

        @pl.when(ch < nchunks)
        def _():
            row = ch * zrows
            pltpu.sync_copy(acc.at[pl.ds(row, zrows)], s_hbm.at[pl.ds(cid * n + row, zrows)])


def _sc_scatter(mg, dst, n):
    e_total, h_dim = mg.shape
    mesh = plsc.VectorSubcoreMesh(core_axis_name="c", subcore_axis_name="s")
    return pl.kernel(
        _sc_scatter_body,
        out_type=jax.ShapeDtypeStruct((2 * n, h_dim), F32),
        mesh=mesh,
        compiler_params=pltpu.CompilerParams(needs_layout_passes=False),
        scratch_types=[
            pltpu.VMEM_SHARED((n, h_dim), F32),
            pltpu.VMEM((_CK, h_dim), F32),
            pltpu.VMEM((_CK // 16, 16), jnp.int32),
            pltpu.VMEM((1, _CK), jnp.int32),
        ],
    )(mg, dst)


def _tail_body(h_ref, s_ref, c_ref, batch_ref,
               uw1, ub1, uw2, ub2, dw1, db1, dw2, db2,
               hw1, hb1, hw2, hb2, out_ref):
    h = h_ref[...]
    n = h.shape[0]
    sfull = s_ref[...]
    s = sfull[:n] + sfull[n:]
    cfull = c_ref[...]
    cnt = lax.dot_general(cfull, jnp.ones((cfull.shape[0], 1), F32),
                          (((0,), (0,)), ((), ())),
                          preferred_element_type=F32)
    agg = s / jnp.maximum(cnt, 1.0)
    upd_in = jnp.concatenate([h, agg], axis=-1)
    t = jnp.maximum(jnp.dot(upd_in, uw1[...], preferred_element_type=F32) + ub1[...], 0.0)
    h = h + jnp.dot(t, uw2[...], preferred_element_type=F32) + ub2[...]
    t = jnp.maximum(jnp.dot(h, dw1[...], preferred_element_type=F32) + db1[...], 0.0)
    hd = jnp.dot(t, dw2[...], preferred_element_type=F32) + db2[...]
    nb = out_ref.shape[0]
    biota = lax.broadcasted_iota(jnp.int32, (nb, 1), 0)
    oht = (biota == batch_ref[...]).astype(F32)
    pooled = jnp.dot(oht, hd, preferred_element_type=F32)
    t = jnp.maximum(jnp.dot(pooled, hw1[...], preferred_element_type=F32) + hb1[...], 0.0)
    out_ref[...] = jnp.dot(t, hw2[...], preferred_element_type=F32) + hb2[...]


def _tail(h, s_part, c_part, batch_row, nb,
          uw1, ub1, uw2, ub2, dw1, db1, dw2, db2, hw1, hb1, hw2, hb2):
    out_dim = hw2.shape[1]
    return pl.pallas_call(
        _tail_body,
        out_shape=jax.ShapeDtypeStruct((nb, out_dim), F32),
    )(h, s_part, c_part, batch_row,
      uw1, ub1.reshape(1, -1), uw2, ub2.reshape(1, -1),
      dw1, db1.reshape(1, -1), dw2, db2.reshape(1, -1),
      hw1, hb1.reshape(1, -1), hw2, hb2.reshape(1, -1))


def kernel(x, pos, edge_index, batch, emb_W1, emb_b1, emb_W2, emb_b2,
           msg_W1, msg_b1, msg_W2, msg_b2, edge_W, edge_b,
           upd_W1, upd_b1, upd_W2, upd_b2, dec_W1, dec_b1, dec_W2, dec_b2,
           head_W1, head_b1, head_W2, head_b2):
    n, _ = x.shape
    h_dim = emb_W1.shape[1]
    nb = 64
    src = edge_index[0].astype(jnp.int32)
    dst = edge_index[1].astype(jnp.int32)
    pos4 = jnp.pad(pos.astype(F32), ((0, 0), (0, 1))).reshape(-1)
    w1i = msg_W1[:h_dim]
    w1j = msg_W1[h_dim:2 * h_dim]
    w1c = msg_W1[2 * h_dim]

    h, a, b = _prep(x, emb_W1, emb_b1, emb_W2, emb_b2, w1i, w1j, msg_b1)
    t, c_flat = _sc_gather(a, b, pos4, src, dst, w1c)
    c_part = c_flat.reshape(32, n)
    mg = _edge_mlp(t, msg_W2, msg_b2, edge_W, edge_b)
    s_part = _sc_scatter(mg, dst, n)
    batch_row = batch.astype(jnp.int32).reshape(1, n)
    return _tail(h, s_part, c_part, batch_row, nb,
                 upd_W1, upd_b1, upd_W2, upd_b2,
                 dec_W1, dec_b1, dec_W2, dec_b2,
                 head_W1, head_b1, head_W2, head_b2)


# speedup vs baseline: 4.9007x; 1.1353x over previous
import functools

import jax
import jax.numpy as jnp
from jax import lax
from jax.experimental import pallas as pl
from jax.experimental.pallas import tpu as pltpu
from jax.experimental.pallas import tpu_sc as plsc

F32 = jnp.float32


def _prep_body(x_ref, ew1, eb1, ew2, eb2, w1i, w1j, b1, h_ref, a_ref, b_ref):
    xx = x_ref[...]
    t = jnp.maximum(jnp.dot(xx, ew1[...], preferred_element_type=F32) + eb1[...], 0.0)
    h = jnp.dot(t, ew2[...], preferred_element_type=F32) + eb2[...]
    h_ref[...] = h
    a_ref[...] = jnp.dot(h, w1i[...], preferred_element_type=F32) + b1[...]
    b_ref[...] = jnp.dot(h, w1j[...], preferred_element_type=F32)


def _prep(x, ew1, eb1, ew2, eb2, w1i, w1j, b1):
    n, f = x.shape
    h_dim = ew1.shape[1]
    blk = 1000
    grid = n // blk
    wspec = lambda shape: pl.BlockSpec(shape, lambda i: (0, 0))
    return pl.pallas_call(
        _prep_body,
        grid=(grid,),
        in_specs=[
            pl.BlockSpec((blk, f), lambda i: (i, 0)),
            wspec(ew1.shape), wspec((1, h_dim)), wspec(ew2.shape), wspec((1, h_dim)),
            wspec(w1i.shape), wspec(w1j.shape), wspec((1, h_dim)),
        ],
        out_specs=[
            pl.BlockSpec((blk, h_dim), lambda i: (i, 0)),
            pl.BlockSpec((blk, h_dim), lambda i: (i, 0)),
            pl.BlockSpec((blk, h_dim), lambda i: (i, 0)),
        ],
        out_shape=[
            jax.ShapeDtypeStruct((n, h_dim), F32),
            jax.ShapeDtypeStruct((n, h_dim), F32),
            jax.ShapeDtypeStruct((n, h_dim), F32),
        ],
    )(x, ew1, eb1.reshape(1, -1), ew2, eb2.reshape(1, -1), w1i, w1j, b1.reshape(1, -1))


_CK = 80


def _sc_gather_body(a_hbm, b_hbm, pos_hbm, src_hbm, dst_hbm, w1c_hbm,
                    t_hbm, c_hbm,
                    posv, w1cv, idxs0, idxs1, idxd0, idxd1,
                    bufa0, bufa1, bufb0, bufb1, distv, cntv,
                    semis0, semis1, semid0, semid1,
                    sema0, sema1, semb0, semb1, semt0, semt1):
    num_cores = 2
    num_sub = 16
    cid = lax.axis_index("c")
    sid = lax.axis_index("s")
    wid = sid * num_cores + cid
    e_total = t_hbm.shape[0]
    epw = e_total // (num_cores * num_sub)
    nchunks = epw // _CK
    base0 = wid * epw

    idxs = [idxs0, idxs1]
    idxd = [idxd0, idxd1]
    bufa = [bufa0, bufa1]
    bufb = [bufb0, bufb1]
    semis = [semis0, semis1]
    semid = [semid0, semid1]
    sema = [sema0, sema1]
    semb = [semb0, semb1]
    semt = [semt0, semt1]

    pltpu.sync_copy(pos_hbm, posv)
    pltpu.sync_copy(w1c_hbm, w1cv)
    w1 = [w1cv[pl.ds(r * 16, 16)] for r in range(8)]
    n_nodes = a_hbm.shape[0]
    zero16 = jnp.zeros((16,), F32)
    ones16 = jnp.full((16,), 1.0, F32)

    def czl(i, carry):
        cntv[pl.ds(i * 16, 16)] = zero16
        return carry

    lax.fori_loop(0, n_nodes // 16, czl, 0)

    def start_idx(base, p):
        pltpu.async_copy(src_hbm.at[pl.ds(base, _CK)], idxs[p], semis[p])
        pltpu.async_copy(dst_hbm.at[pl.ds(base, _CK)], idxd[p], semid[p])

    def do_chunk(c, p, wait_t, start_next):
        base = base0 + c * _CK
        pltpu.make_async_copy(src_hbm.at[pl.ds(base, _CK)], idxs[p], semis[p]).wait()
        pltpu.make_async_copy(dst_hbm.at[pl.ds(base, _CK)], idxd[p], semid[p]).wait()
        if wait_t:
            pltpu.make_async_copy(bufa[p], t_hbm.at[pl.ds(base, _CK)], semt[p]).wait()
        cpa = pltpu.async_copy(a_hbm.at[idxd[p]], bufa[p], sema[p])
        cpb = pltpu.async_copy(b_hbm.at[idxs[p]], bufb[p], semb[p])
        if start_next:
            start_idx(base + _CK, 1 - p)

        def dloop(j, carry2):
            d16raw = idxd[p][pl.ds(j * 16, 16)]
            plsc.addupdate_scatter(cntv, [d16raw], ones16)
            s16 = idxs[p][pl.ds(j * 16, 16)] * 4
            d16 = d16raw * 4
            dx = plsc.load_gather(posv, [s16]) - plsc.load_gather(posv, [d16])
            dy = plsc.load_gather(posv, [s16 + 1]) - plsc.load_gather(posv, [d16 + 1])
            dz = plsc.load_gather(posv, [s16 + 2]) - plsc.load_gather(posv, [d16 + 2])
            dd = dx * dx + dy * dy + dz * dz
            ii = jnp.int32(0x5F3759DF) - lax.shift_right_logical(
                plsc.bitcast(dd, jnp.int32), 1)
            y = plsc.bitcast(ii, F32)
            y = y * (1.5 - 0.5 * dd * y * y)
            y = y * (1.5 - 0.5 * dd * y * y)
            y = y * (1.5 - 0.5 * dd * y * y)
            distv[pl.ds(j * 16, 16)] = dd * y
            return carry2

        lax.fori_loop(0, _CK // 16, dloop, 0)
        cpa.wait()
        cpb.wait()

        def gloop(g, carry2):
            dvec = distv[pl.ds(g * 16, 16)]
            for lane in range(16):
                dsc = dvec[lane]
                e = g * 16 + lane
                for r in range(8):
                    col = pl.ds(r * 16, 16)
                    tv = bufa[p][e, col] + bufb[p][e, col] + dsc * w1[r]
                    bufa[p][e, col] = jnp.maximum(tv, 0.0)
            return carry2

        lax.fori_loop(0, _CK // 16, gloop, 0)
        pltpu.async_copy(bufa[p], t_hbm.at[pl.ds(base, _CK)], semt[p])

    start_idx(base0, 0)
    do_chunk(0, 0, wait_t=False, start_next=True)
    do_chunk(1, 1, wait_t=False, start_next=True)

    def pair(i, carry):
        c = 2 * i
        do_chunk(c, 0, wait_t=True, start_next=True)
        do_chunk(c + 1, 1, wait_t=True, start_next=True)
        return carry

    lax.fori_loop(1, nchunks // 2, pair, 0)
    do_chunk(nchunks - 1, 0, wait_t=True, start_next=False)
    pltpu.make_async_copy(bufa[1], t_hbm.at[pl.ds(base0, _CK)], semt[1]).wait()
    pltpu.make_async_copy(bufa[0], t_hbm.at[pl.ds(base0, _CK)], semt[0]).wait()
    pltpu.sync_copy(cntv, c_hbm.at[pl.ds(wid * n_nodes, n_nodes)])


def _sc_gather(a, b, pos4, src, dst, w1c):
    n, h_dim = a.shape
    e_total = src.shape[0]
    mesh = plsc.VectorSubcoreMesh(core_axis_name="c", subcore_axis_name="s")
    return pl.kernel(
        _sc_gather_body,
        out_type=[
            jax.ShapeDtypeStruct((e_total, h_dim), F32),
            jax.ShapeDtypeStruct((32 * n,), F32),
        ],
        mesh=mesh,
        compiler_params=pltpu.CompilerParams(needs_layout_passes=False),
        scratch_types=[
            pltpu.VMEM((n * 4,), F32),
            pltpu.VMEM((h_dim,), F32),
            pltpu.VMEM((_CK,), jnp.int32),
            pltpu.VMEM((_CK,), jnp.int32),
            pltpu.VMEM((_CK,), jnp.int32),
            pltpu.VMEM((_CK,), jnp.int32),
            pltpu.VMEM((_CK, h_dim), F32),
            pltpu.VMEM((_CK, h_dim), F32),
            pltpu.VMEM((_CK, h_dim), F32),
            pltpu.VMEM((_CK, h_dim), F32),
            pltpu.VMEM((_CK,), F32),
            pltpu.VMEM((n,), F32),
        ] + [pltpu.SemaphoreType.DMA] * 10,
    )(a, b, pos4, src, dst, w1c)


def _edge_body(t_ref, w2, b2, ew, eb, mg_ref):
    t = t_ref[...]
    m = jnp.maximum(jnp.dot(t, w2[...], preferred_element_type=F32) + b2[...], 0.0)
    g = jnp.dot(m, ew[...], preferred_element_type=F32) + eb[...]
    g = 1.0 / (1.0 + jnp.exp(-g))
    mg_ref[...] = m * g


def _edge_mlp(t, w2, b2, ew, eb):
    e_total, h_dim = t.shape
    blk = 3200
    grid = e_total // blk
    wspec = lambda shape: pl.BlockSpec(shape, lambda i: (0, 0))
    return pl.pallas_call(
        _edge_body,
        grid=(grid,),
        in_specs=[
            pl.BlockSpec((blk, h_dim), lambda i: (i, 0)),
            wspec(w2.shape), wspec((1, h_dim)), wspec(ew.shape), wspec((1, 1)),
        ],
        out_specs=pl.BlockSpec((blk, h_dim), lambda i: (i, 0)),
        out_shape=jax.ShapeDtypeStruct((e_total, h_dim), F32),
    )(t, w2, b2.reshape(1, -1), ew, eb.reshape(1, 1))


def _sc_scatter_body(mg_hbm, dst_hbm, s_hbm,
                     acc, bufm0, bufm1, idxd0, idxd1, idxw0, idxw1,
                     semm0, semm1, semi0, semi1, semsc0, semsc1):
    num_cores = 2
    num_sub = 16
    cid = lax.axis_index("c")
    sid = lax.axis_index("s")
    e_total = mg_hbm.shape[0]
    n = acc.shape[0]
    zrows = _CK
    nchunks = n // zrows
    epc = e_total // num_cores
    ept = epc // num_sub
    nec = ept // _CK
    nsub = _CK // 16

    bufm = [bufm0, bufm1]
    idxd = [idxd0, idxd1]
    idxw = [idxw0, idxw1]
    semm = [semm0, semm1]
    semi = [semi0, semi1]
    semsc = [semsc0, semsc1]

    zero = jnp.zeros((16,), F32)

    def zloop(r, carry):
        for q in range(8):
            bufm0[r, pl.ds(q * 16, 16)] = zero
        return carry

    lax.fori_loop(0, zrows, zloop, 0)

    nround = (nchunks + num_sub - 1) // num_sub
    for q in range(nround):
        ch = sid + q * num_sub

        @pl.when(ch < nchunks)
        def _():
            pltpu.sync_copy(bufm0, acc.at[pl.ds(ch * zrows, zrows)])
    plsc.subcore_barrier()

    base0 = cid * epc + sid * ept

    def start_loads(base, p):
        pltpu.async_copy(dst_hbm.at[pl.ds(base, _CK)], idxw[p].at[0], semi[p])
        pltpu.async_copy(mg_hbm.at[pl.ds(base, _CK)], bufm[p], semm[p])

    def drain_scatters(p):
        for k in range(nsub):
            pltpu.make_async_copy(bufm[p].at[pl.ds(k * 16, 16)],
                                  acc.at[idxd[p].at[k]], semsc[p]).wait()

    def do_chunk(c, p, drain_prev, start_next):
        base = base0 + c * _CK
        pltpu.make_async_copy(dst_hbm.at[pl.ds(base, _CK)], idxw[p].at[0],
                              semi[p]).wait()
        pltpu.make_async_copy(mg_hbm.at[pl.ds(base, _CK)], bufm[p],
                              semm[p]).wait()
        if drain_prev:
            drain_scatters(1 - p)
        if start_next:
            start_loads(base + _CK, 1 - p)
        for k in range(nsub):
            idxd[p][k, pl.ds(0, 16)] = idxw[p][0, pl.ds(k * 16, 16)]
        for k in range(nsub):
            pltpu.async_copy(bufm[p].at[pl.ds(k * 16, 16)],
                             acc.at[idxd[p].at[k]], semsc[p], add=True)

    start_loads(base0, 0)
    do_chunk(0, 0, drain_prev=False, start_next=True)

    def pair(i, carry):
        c = 2 * i
        do_chunk(c + 1, 1, drain_prev=True, start_next=True)
        do_chunk(c + 2, 0, drain_prev=True, start_next=True)
        return carry

    lax.fori_loop(0, (nec - 3) // 2, pair, 0)
    do_chunk(nec - 2, 1, drain_prev=True, start_next=True)
    do_chunk(nec - 1, 0, drain_prev=True, start_next=False)
    drain_scatters(0)
    plsc.subcore_barrier()

    for q in range(nround):
        ch = sid + q * num_sub

        @pl.when(ch < nchunks)
        def _():
            row = ch * zrows
            pltpu.sync_copy(acc.at[pl.ds(row, zrows)], s_hbm.at[pl.ds(cid * n + row, zrows)])


def _sc_scatter(mg, dst, n):
    e_total, h_dim = mg.shape
    mesh = plsc.VectorSubcoreMesh(core_axis_name="c", subcore_axis_name="s")
    return pl.kernel(
        _sc_scatter_body,
        out_type=jax.ShapeDtypeStruct((2 * n, h_dim), F32),
        mesh=mesh,
        compiler_params=pltpu.CompilerParams(needs_layout_passes=False),
        scratch_types=[
            pltpu.VMEM_SHARED((n, h_dim), F32),
            pltpu.VMEM((_CK, h_dim), F32),
            pltpu.VMEM((_CK, h_dim), F32),
            pltpu.VMEM((_CK // 16, 16), jnp.int32),
            pltpu.VMEM((_CK // 16, 16), jnp.int32),
            pltpu.VMEM((1, _CK), jnp.int32),
            pltpu.VMEM((1, _CK), jnp.int32),
        ] + [pltpu.SemaphoreType.DMA] * 6,
    )(mg, dst)


def _tail_body(h_ref, s_ref, c_ref, batch_ref,
               uw1, ub1, uw2, ub2, dw1, db1, dw2, db2,
               hw1, hb1, hw2, hb2, out_ref):
    h = h_ref[...]
    n = h.shape[0]
    sfull = s_ref[...]
    s = sfull[:n] + sfull[n:]
    cfull = c_ref[...]
    cnt = lax.dot_general(cfull, jnp.ones((cfull.shape[0], 1), F32),
                          (((0,), (0,)), ((), ())),
                          preferred_element_type=F32)
    agg = s / jnp.maximum(cnt, 1.0)
    upd_in = jnp.concatenate([h, agg], axis=-1)
    t = jnp.maximum(jnp.dot(upd_in, uw1[...], preferred_element_type=F32) + ub1[...], 0.0)
    h = h + jnp.dot(t, uw2[...], preferred_element_type=F32) + ub2[...]
    t = jnp.maximum(jnp.dot(h, dw1[...], preferred_element_type=F32) + db1[...], 0.0)
    hd = jnp.dot(t, dw2[...], preferred_element_type=F32) + db2[...]
    nb = out_ref.shape[0]
    biota = lax.broadcasted_iota(jnp.int32, (nb, 1), 0)
    oht = (biota == batch_ref[...]).astype(F32)
    pooled = jnp.dot(oht, hd, preferred_element_type=F32)
    t = jnp.maximum(jnp.dot(pooled, hw1[...], preferred_element_type=F32) + hb1[...], 0.0)
    out_ref[...] = jnp.dot(t, hw2[...], preferred_element_type=F32) + hb2[...]


def _tail(h, s_part, c_part, batch_row, nb,
          uw1, ub1, uw2, ub2, dw1, db1, dw2, db2, hw1, hb1, hw2, hb2):
    out_dim = hw2.shape[1]
    return pl.pallas_call(
        _tail_body,
        out_shape=jax.ShapeDtypeStruct((nb, out_dim), F32),
    )(h, s_part, c_part, batch_row,
      uw1, ub1.reshape(1, -1), uw2, ub2.reshape(1, -1),
      dw1, db1.reshape(1, -1), dw2, db2.reshape(1, -1),
      hw1, hb1.reshape(1, -1), hw2, hb2.reshape(1, -1))


def kernel(x, pos, edge_index, batch, emb_W1, emb_b1, emb_W2, emb_b2,
           msg_W1, msg_b1, msg_W2, msg_b2, edge_W, edge_b,
           upd_W1, upd_b1, upd_W2, upd_b2, dec_W1, dec_b1, dec_W2, dec_b2,
           head_W1, head_b1, head_W2, head_b2):
    n, _ = x.shape
    h_dim = emb_W1.shape[1]
    nb = 64
    src = edge_index[0].astype(jnp.int32)
    dst = edge_index[1].astype(jnp.int32)
    pos4 = jnp.pad(pos.astype(F32), ((0, 0), (0, 1))).reshape(-1)
    w1i = msg_W1[:h_dim]
    w1j = msg_W1[h_dim:2 * h_dim]
    w1c = msg_W1[2 * h_dim]

    h, a, b = _prep(x, emb_W1, emb_b1, emb_W2, emb_b2, w1i, w1j, msg_b1)
    t, c_flat = _sc_gather(a, b, pos4, src, dst, w1c)
    c_part = c_flat.reshape(32, n)
    mg = _edge_mlp(t, msg_W2, msg_b2, edge_W, edge_b)
    s_part = _sc_scatter(mg, dst, n)
    batch_row = batch.astype(jnp.int32).reshape(1, n)
    return _tail(h, s_part, c_part, batch_row, nb,
                 upd_W1, upd_b1, upd_W2, upd_b2,
                 dec_W1, dec_b1, dec_W2, dec_b2,
                 head_W1, head_b1, head_W2, head_b2)
